# baseline (speedup 1.0000x reference)
import jax
import jax.numpy as jnp
from jax import lax
from jax.experimental import pallas as pl
from jax.experimental.pallas import tpu as pltpu
from jax.experimental.pallas import tpu_sc as plsc

N = 10000
E = 320000
D = 128
NC, NS = 2, 16
NW = NC * NS
E_PER_W = E // NW
CHUNK = 80
NCH = E_PER_W // CHUNK
DEG_W = 16
R = 1000
ROWS_A = 632
ROWS_B = N - (NS - 1) * ROWS_A


def _vmesh():
    return plsc.VectorSubcoreMesh(core_axis_name="c", subcore_axis_name="s")


def _for_tile_rows(s, fn):
    @pl.when(s < NS - 1)
    def _():
        fn(pl.ds(pl.multiple_of(s * ROWS_A, 8), ROWS_A))

    @pl.when(s == NS - 1)
    def _():
        fn(pl.ds((NS - 1) * ROWS_A, ROWS_B))


def _sc_degree(dst1, init2):
    out_t = (jax.ShapeDtypeStruct((N, DEG_W), jnp.float32),) * 2

    NSLOT = 4

    @pl.kernel(out_type=out_t, mesh=_vmesh(),
               scratch_types=[pltpu.VMEM_SHARED((N, DEG_W), jnp.float32),
                              pltpu.VMEM((NSLOT, CHUNK), jnp.int32),
                              pltpu.VMEM((CHUNK, DEG_W), jnp.float32),
                              [pltpu.SemaphoreType.DMA] * NSLOT,
                              [pltpu.SemaphoreType.DMA] * NSLOT])
    def k(dst_hbm, init_hbm, oA, oB, acc, didx, onesb, dsems, ssems):
        c = lax.axis_index("c")
        s = lax.axis_index("s")
        wid = c * NS + s

        def d_async(j, slot):
            return pltpu.async_copy(
                dst_hbm.at[pl.ds(wid * E_PER_W + j * CHUNK, CHUNK)],
                didx.at[slot], dsems[slot])

        _for_tile_rows(s, lambda sl: pltpu.sync_copy(init_hbm.at[c].at[sl],
                                                     acc.at[sl]))
        pltpu.sync_copy(init_hbm.at[0].at[pl.ds(0, CHUNK)], onesb)
        plsc.subcore_barrier()

        def group(j, nslot):
            ds = [d_async(j + i, i) for i in range(nslot)]
            ss = []
            for i in range(nslot):
                ds[i].wait()
                ss.append(pltpu.async_copy(onesb, acc.at[didx.at[i]],
                                           ssems[i], add=True))
            for cp in ss:
                cp.wait()

        @pl.loop(0, NCH - 1, step=NSLOT)
        def _(j):
            group(j, NSLOT)

        group(NCH - 1, 1)
        plsc.subcore_barrier()

        @pl.when(c == 0)
        def _():
            _for_tile_rows(s, lambda sl: pltpu.sync_copy(acc.at[sl], oA.at[sl]))

        @pl.when(c == 1)
        def _():
            _for_tile_rows(s, lambda sl: pltpu.sync_copy(acc.at[sl], oB.at[sl]))

    return k(dst1, init2)


def _sc_aggregate(z, zeros_nd, src1, dst1):
    out_t = (jax.ShapeDtypeStruct((N, D), jnp.float32),) * 2
    CH = 40
    NCHU = E_PER_W // CH
    NSLOT = 5

    @pl.kernel(out_type=out_t, mesh=_vmesh(),
               scratch_types=[pltpu.VMEM_SHARED((N, D), jnp.float32),
                              pltpu.VMEM((E_PER_W,), jnp.int32),
                              pltpu.VMEM((NSLOT, CH), jnp.int32),
                              pltpu.VMEM((NSLOT, CH, D), jnp.float32),
                              [pltpu.SemaphoreType.DMA] * NSLOT,
                              [pltpu.SemaphoreType.DMA] * NSLOT,
                              [pltpu.SemaphoreType.DMA] * NSLOT])
    def k(z_hbm, zeros_hbm, src_hbm, dst_hbm, oA, oB,
          acc, sidx, didx, rows, gsems, dsems, ssems):
        c = lax.axis_index("c")
        s = lax.axis_index("s")
        wid = c * NS + s
        base = wid * E_PER_W
        i1 = pltpu.async_copy(src_hbm.at[pl.ds(base, E_PER_W)], sidx, gsems[0])

        def d_async(j, slot):
            return pltpu.async_copy(
                dst_hbm.at[pl.ds(base + j * CH, CH)], didx.at[slot],
                dsems[slot])

        @pl.when(c == 0)
        def _():
            _for_tile_rows(s, lambda sl: pltpu.sync_copy(z_hbm.at[sl],
                                                         acc.at[sl]))

        @pl.when(c == 1)
        def _():
            _for_tile_rows(s, lambda sl: pltpu.sync_copy(zeros_hbm.at[sl],
                                                         acc.at[sl]))

        i1.wait()
        plsc.subcore_barrier()

        def g_async(j, slot):
            return pltpu.async_copy(
                z_hbm.at[sidx.at[pl.ds(j * CH, CH)]], rows.at[slot],
                gsems[slot])

        def s_async(slot):
            return pltpu.async_copy(rows.at[slot], acc.at[didx.at[slot]],
                                    ssems[slot], add=True)

        def group(j, nslot):
            ds = [d_async(j + i, i) for i in range(nslot)]
            gs = [g_async(j + i, i) for i in range(nslot)]
            ss = []
            for i in range(nslot):
                gs[i].wait()
                ds[i].wait()
                ss.append(s_async(i))
            for cp in ss:
                cp.wait()

        @pl.loop(0, NCHU, step=NSLOT)
        def _(j):
            group(j, NSLOT)

        plsc.subcore_barrier()

        @pl.when(c == 0)
        def _():
            _for_tile_rows(s, lambda sl: pltpu.sync_copy(acc.at[sl], oA.at[sl]))

        @pl.when(c == 1)
        def _():
            _for_tile_rows(s, lambda sl: pltpu.sync_copy(acc.at[sl], oB.at[sl]))

    return k(z, zeros_nd, src1, dst1)


def _dinv_block(dA_ref, dB_ref):
    return lax.rsqrt(dA_ref[:, 0:1] + dB_ref[:, 0:1])


def _tc_layer1(x, W1, degA, degB):
    def body(x_ref, w_ref, dA_ref, dB_ref, o_ref):
        dinv = _dinv_block(dA_ref, dB_ref)
        u = jnp.dot(x_ref[...], w_ref[...], preferred_element_type=jnp.float32)
        o_ref[...] = dinv * u

    return pl.pallas_call(
        body,
        grid=(N // R,),
        in_specs=[pl.BlockSpec((R, D), lambda i: (i, 0)),
                  pl.BlockSpec((D, D), lambda i: (0, 0)),
                  pl.BlockSpec((R, DEG_W), lambda i: (i, 0)),
                  pl.BlockSpec((R, DEG_W), lambda i: (i, 0))],
        out_specs=pl.BlockSpec((R, D), lambda i: (i, 0)),
        out_shape=jax.ShapeDtypeStruct((N, D), jnp.float32),
    )(x, W1, degA, degB)


def _tc_layer(sA, sB, degA, degB, b_prev, W):
    def body(sA_ref, sB_ref, dA_ref, dB_ref, b_ref, w_ref, o_ref):
        dinv = _dinv_block(dA_ref, dB_ref)
        h = jnp.maximum(dinv * (sA_ref[...] + sB_ref[...]) + b_ref[...], 0.0)
        u = jnp.dot(h, w_ref[...], preferred_element_type=jnp.float32)
        o_ref[...] = dinv * u

    return pl.pallas_call(
        body,
        grid=(N // R,),
        in_specs=[pl.BlockSpec((R, D), lambda i: (i, 0)),
                  pl.BlockSpec((R, D), lambda i: (i, 0)),
                  pl.BlockSpec((R, DEG_W), lambda i: (i, 0)),
                  pl.BlockSpec((R, DEG_W), lambda i: (i, 0)),
                  pl.BlockSpec((1, D), lambda i: (0, 0)),
                  pl.BlockSpec((D, D), lambda i: (0, 0))],
        out_specs=pl.BlockSpec((R, D), lambda i: (i, 0)),
        out_shape=jax.ShapeDtypeStruct((N, D), jnp.float32),
    )(sA, sB, degA, degB, b_prev.reshape(1, D), W)


def _tc_final(sA, sB, degA, degB, batch2d, b4, Wl, bl, num_graphs):
    G = num_graphs
    n_cls = Wl.shape[1]
    steps = N // R

    def body(sA_ref, sB_ref, dA_ref, dB_ref, bt_ref, b4_ref, wl_ref, bl_ref,
             o_ref, gsum, cnt):
        i = pl.program_id(0)

        @pl.when(i == 0)
        def _():
            gsum[...] = jnp.zeros_like(gsum)
            cnt[...] = jnp.zeros_like(cnt)

        dinv = _dinv_block(dA_ref, dB_ref)
        t = dinv * (sA_ref[...] + sB_ref[...])
        gid = lax.broadcasted_iota(jnp.int32, (1, G), 1)
        oh = (bt_ref[...] == gid).astype(jnp.float32)
        gsum[...] += lax.dot_general(oh, t, (((0,), (0,)), ((), ())),
                                     preferred_element_type=jnp.float32)
        cnt[...] += lax.dot_general(oh, jnp.ones((R, 1), jnp.float32),
                                    (((0,), (0,)), ((), ())),
                                    preferred_element_type=jnp.float32)

        @pl.when(i == steps - 1)
        def _():
            gmean = gsum[...] / jnp.maximum(cnt[...], 1.0)
            o_ref[...] = jnp.dot(gmean + b4_ref[...], wl_ref[...],
                                 preferred_element_type=jnp.float32) + bl_ref[...]

    return pl.pallas_call(
        body,
        grid=(steps,),
        in_specs=[pl.BlockSpec((R, D), lambda i: (i, 0)),
                  pl.BlockSpec((R, D), lambda i: (i, 0)),
                  pl.BlockSpec((R, DEG_W), lambda i: (i, 0)),
                  pl.BlockSpec((R, DEG_W), lambda i: (i, 0)),
                  pl.BlockSpec((R, 1), lambda i: (i, 0)),
                  pl.BlockSpec((1, D), lambda i: (0, 0)),
                  pl.BlockSpec((D, n_cls), lambda i: (0, 0)),
                  pl.BlockSpec((1, n_cls), lambda i: (0, 0))],
        out_specs=pl.BlockSpec((G, n_cls), lambda i: (0, 0)),
        out_shape=jax.ShapeDtypeStruct((G, n_cls), jnp.float32),
        scratch_shapes=[pltpu.VMEM((G, D), jnp.float32),
                        pltpu.VMEM((G, 1), jnp.float32)],
    )(sA, sB, degA, degB, batch2d, b4.reshape(1, D), Wl,
      bl.reshape(1, n_cls))


def kernel(x, edge_index, batch, y, W1, b1, W2, b2, W3, b3, W4, b4, Wl, bl):
    src1 = edge_index[0]
    dst1 = edge_index[1]
    num_graphs = y.shape[0]
    init2 = jnp.concatenate([jnp.ones((1, N, DEG_W), jnp.float32),
                             jnp.zeros((1, N, DEG_W), jnp.float32)], axis=0)
    zeros_nd = jnp.zeros((N, D), jnp.float32)

    degA, degB = _sc_degree(dst1, init2)
    z = _tc_layer1(x, W1, degA, degB)
    sA, sB = _sc_aggregate(z, zeros_nd, src1, dst1)
    z = _tc_layer(sA, sB, degA, degB, b1, W2)
    sA, sB = _sc_aggregate(z, zeros_nd, src1, dst1)
    z = _tc_layer(sA, sB, degA, degB, b2, W3)
    sA, sB = _sc_aggregate(z, zeros_nd, src1, dst1)
    z = _tc_layer(sA, sB, degA, degB, b3, W4)
    sA, sB = _sc_aggregate(z, zeros_nd, src1, dst1)
    return _tc_final(sA, sB, degA, degB, batch.reshape(N, 1), b4, Wl, bl,
                     num_graphs)


# speedup vs baseline: 1.0370x; 1.0370x over previous
import jax
import jax.numpy as jnp
from jax import lax
from jax.experimental import pallas as pl
from jax.experimental.pallas import tpu as pltpu
from jax.experimental.pallas import tpu_sc as plsc

N = 10000
E = 320000
D = 128
NC, NS = 2, 16
NW = NC * NS
E_PER_W = E // NW
CHUNK = 80
NCH = E_PER_W // CHUNK
DEG_W = 16
R = 1000
ROWS_A = 632
ROWS_B = N - (NS - 1) * ROWS_A


def _vmesh():
    return plsc.VectorSubcoreMesh(core_axis_name="c", subcore_axis_name="s")


def _for_tile_rows(s, fn):
    @pl.when(s < NS - 1)
    def _():
        fn(pl.ds(pl.multiple_of(s * ROWS_A, 8), ROWS_A))

    @pl.when(s == NS - 1)
    def _():
        fn(pl.ds((NS - 1) * ROWS_A, ROWS_B))


def _sc_degree(dst1, init2):
    out_t = (jax.ShapeDtypeStruct((N, DEG_W), jnp.float32),) * 2

    NSLOT = 4

    @pl.kernel(out_type=out_t, mesh=_vmesh(),
               scratch_types=[pltpu.VMEM_SHARED((N, DEG_W), jnp.float32),
                              pltpu.VMEM((NSLOT, CHUNK), jnp.int32),
                              pltpu.VMEM((CHUNK, DEG_W), jnp.float32),
                              [pltpu.SemaphoreType.DMA] * NSLOT,
                              [pltpu.SemaphoreType.DMA] * NSLOT])
    def k(dst_hbm, init_hbm, oA, oB, acc, didx, onesb, dsems, ssems):
        c = lax.axis_index("c")
        s = lax.axis_index("s")
        wid = c * NS + s

        def d_async(j, slot):
            return pltpu.async_copy(
                dst_hbm.at[pl.ds(wid * E_PER_W + j * CHUNK, CHUNK)],
                didx.at[slot], dsems[slot])

        _for_tile_rows(s, lambda sl: pltpu.sync_copy(init_hbm.at[c].at[sl],
                                                     acc.at[sl]))
        pltpu.sync_copy(init_hbm.at[0].at[pl.ds(0, CHUNK)], onesb)
        plsc.subcore_barrier()

        def group(j, nslot):
            ds = [d_async(j + i, i) for i in range(nslot)]
            ss = []
            for i in range(nslot):
                ds[i].wait()
                ss.append(pltpu.async_copy(onesb, acc.at[didx.at[i]],
                                           ssems[i], add=True))
            for cp in ss:
                cp.wait()

        @pl.loop(0, NCH - 1, step=NSLOT)
        def _(j):
            group(j, NSLOT)

        group(NCH - 1, 1)
        plsc.subcore_barrier()

        @pl.when(c == 0)
        def _():
            _for_tile_rows(s, lambda sl: pltpu.sync_copy(acc.at[sl], oA.at[sl]))

        @pl.when(c == 1)
        def _():
            _for_tile_rows(s, lambda sl: pltpu.sync_copy(acc.at[sl], oB.at[sl]))

    return k(dst1, init2)


def _sc_aggregate(z, zeros_nd, src1, dst1):
    out_t = (jax.ShapeDtypeStruct((N, D), jnp.float32),) * 2
    CH = 40
    NCHU = E_PER_W // CH
    NSLOT = 4

    @pl.kernel(out_type=out_t, mesh=_vmesh(),
               scratch_types=[pltpu.VMEM_SHARED((N, D), jnp.float32),
                              pltpu.VMEM((E_PER_W,), jnp.int32),
                              pltpu.VMEM((NSLOT, CH), jnp.int32),
                              pltpu.VMEM((NSLOT, CH, D), jnp.float32),
                              [pltpu.SemaphoreType.DMA] * NSLOT,
                              [pltpu.SemaphoreType.DMA] * NSLOT,
                              [pltpu.SemaphoreType.DMA] * NSLOT])
    def k(z_hbm, zeros_hbm, src_hbm, dst_hbm, oA, oB,
          acc, sidx, didx, rows, gsems, dsems, ssems):
        c = lax.axis_index("c")
        s = lax.axis_index("s")
        wid = c * NS + s
        base = wid * E_PER_W
        i1 = pltpu.async_copy(src_hbm.at[pl.ds(base, E_PER_W)], sidx, gsems[0])

        def d_async(j, slot):
            return pltpu.async_copy(
                dst_hbm.at[pl.ds(base + j * CH, CH)], didx.at[slot],
                dsems[slot])

        @pl.when(c == 0)
        def _():
            _for_tile_rows(s, lambda sl: pltpu.sync_copy(z_hbm.at[sl],
                                                         acc.at[sl]))

        @pl.when(c == 1)
        def _():
            _for_tile_rows(s, lambda sl: pltpu.sync_copy(zeros_hbm.at[sl],
                                                         acc.at[sl]))

        i1.wait()
        plsc.subcore_barrier()

        def g_async(j, slot):
            return pltpu.async_copy(
                z_hbm.at[sidx.at[pl.ds(j * CH, CH)]], rows.at[slot],
                gsems[slot])

        def s_async(slot):
            return pltpu.async_copy(rows.at[slot], acc.at[didx.at[slot]],
                                    ssems[slot], add=True)

        def group(j, nslot):
            ds = [d_async(j + i, i) for i in range(nslot)]
            gs = [g_async(j + i, i) for i in range(nslot)]
            ss = []
            for i in range(nslot):
                gs[i].wait()
                ds[i].wait()
                ss.append(s_async(i))
            for cp in ss:
                cp.wait()

        @pl.loop(0, NCHU - 2, step=NSLOT)
        def _(j):
            group(j, NSLOT)

        group(NCHU - 2, 2)
        plsc.subcore_barrier()

        @pl.when(c == 0)
        def _():
            _for_tile_rows(s, lambda sl: pltpu.sync_copy(acc.at[sl], oA.at[sl]))

        @pl.when(c == 1)
        def _():
            _for_tile_rows(s, lambda sl: pltpu.sync_copy(acc.at[sl], oB.at[sl]))

    return k(z, zeros_nd, src1, dst1)


def _dinv_block(dA_ref, dB_ref):
    return lax.rsqrt(dA_ref[:, 0:1] + dB_ref[:, 0:1])


def _tc_layer1(x, W1, degA, degB):
    def body(x_ref, w_ref, dA_ref, dB_ref, o_ref):
        dinv = _dinv_block(dA_ref, dB_ref)
        u = jnp.dot(x_ref[...], w_ref[...], preferred_element_type=jnp.float32)
        o_ref[...] = dinv * u

    return pl.pallas_call(
        body,
        grid=(N // R,),
        in_specs=[pl.BlockSpec((R, D), lambda i: (i, 0)),
                  pl.BlockSpec((D, D), lambda i: (0, 0)),
                  pl.BlockSpec((R, DEG_W), lambda i: (i, 0)),
                  pl.BlockSpec((R, DEG_W), lambda i: (i, 0))],
        out_specs=pl.BlockSpec((R, D), lambda i: (i, 0)),
        out_shape=jax.ShapeDtypeStruct((N, D), jnp.float32),
    )(x, W1, degA, degB)


def _tc_layer(sA, sB, degA, degB, b_prev, W):
    def body(sA_ref, sB_ref, dA_ref, dB_ref, b_ref, w_ref, o_ref):
        dinv = _dinv_block(dA_ref, dB_ref)
        h = jnp.maximum(dinv * (sA_ref[...] + sB_ref[...]) + b_ref[...], 0.0)
        u = jnp.dot(h, w_ref[...], preferred_element_type=jnp.float32)
        o_ref[...] = dinv * u

    return pl.pallas_call(
        body,
        grid=(N // R,),
        in_specs=[pl.BlockSpec((R, D), lambda i: (i, 0)),
                  pl.BlockSpec((R, D), lambda i: (i, 0)),
                  pl.BlockSpec((R, DEG_W), lambda i: (i, 0)),
                  pl.BlockSpec((R, DEG_W), lambda i: (i, 0)),
                  pl.BlockSpec((1, D), lambda i: (0, 0)),
                  pl.BlockSpec((D, D), lambda i: (0, 0))],
        out_specs=pl.BlockSpec((R, D), lambda i: (i, 0)),
        out_shape=jax.ShapeDtypeStruct((N, D), jnp.float32),
    )(sA, sB, degA, degB, b_prev.reshape(1, D), W)


def _tc_final(sA, sB, degA, degB, batch2d, b4, Wl, bl, num_graphs):
    G = num_graphs
    n_cls = Wl.shape[1]
    steps = N // R

    def body(sA_ref, sB_ref, dA_ref, dB_ref, bt_ref, b4_ref, wl_ref, bl_ref,
             o_ref, gsum, cnt):
        i = pl.program_id(0)

        @pl.when(i == 0)
        def _():
            gsum[...] = jnp.zeros_like(gsum)
            cnt[...] = jnp.zeros_like(cnt)

        dinv = _dinv_block(dA_ref, dB_ref)
        t = dinv * (sA_ref[...] + sB_ref[...])
        gid = lax.broadcasted_iota(jnp.int32, (1, G), 1)
        oh = (bt_ref[...] == gid).astype(jnp.float32)
        gsum[...] += lax.dot_general(oh, t, (((0,), (0,)), ((), ())),
                                     preferred_element_type=jnp.float32)
        cnt[...] += lax.dot_general(oh, jnp.ones((R, 1), jnp.float32),
                                    (((0,), (0,)), ((), ())),
                                    preferred_element_type=jnp.float32)

        @pl.when(i == steps - 1)
        def _():
            gmean = gsum[...] / jnp.maximum(cnt[...], 1.0)
            o_ref[...] = jnp.dot(gmean + b4_ref[...], wl_ref[...],
                                 preferred_element_type=jnp.float32) + bl_ref[...]

    return pl.pallas_call(
        body,
        grid=(steps,),
        in_specs=[pl.BlockSpec((R, D), lambda i: (i, 0)),
                  pl.BlockSpec((R, D), lambda i: (i, 0)),
                  pl.BlockSpec((R, DEG_W), lambda i: (i, 0)),
                  pl.BlockSpec((R, DEG_W), lambda i: (i, 0)),
                  pl.BlockSpec((R, 1), lambda i: (i, 0)),
                  pl.BlockSpec((1, D), lambda i: (0, 0)),
                  pl.BlockSpec((D, n_cls), lambda i: (0, 0)),
                  pl.BlockSpec((1, n_cls), lambda i: (0, 0))],
        out_specs=pl.BlockSpec((G, n_cls), lambda i: (0, 0)),
        out_shape=jax.ShapeDtypeStruct((G, n_cls), jnp.float32),
        scratch_shapes=[pltpu.VMEM((G, D), jnp.float32),
                        pltpu.VMEM((G, 1), jnp.float32)],
    )(sA, sB, degA, degB, batch2d, b4.reshape(1, D), Wl,
      bl.reshape(1, n_cls))


def kernel(x, edge_index, batch, y, W1, b1, W2, b2, W3, b3, W4, b4, Wl, bl):
    src1 = edge_index[0]
    dst1 = edge_index[1]
    num_graphs = y.shape[0]
    init2 = jnp.concatenate([jnp.ones((1, N, DEG_W), jnp.float32),
                             jnp.zeros((1, N, DEG_W), jnp.float32)], axis=0)
    zeros_nd = jnp.zeros((N, D), jnp.float32)

    degA, degB = _sc_degree(dst1, init2)
    z = _tc_layer1(x, W1, degA, degB)
    sA, sB = _sc_aggregate(z, zeros_nd, src1, dst1)
    z = _tc_layer(sA, sB, degA, degB, b1, W2)
    sA, sB = _sc_aggregate(z, zeros_nd, src1, dst1)
    z = _tc_layer(sA, sB, degA, degB, b2, W3)
    sA, sB = _sc_aggregate(z, zeros_nd, src1, dst1)
    z = _tc_layer(sA, sB, degA, degB, b3, W4)
    sA, sB = _sc_aggregate(z, zeros_nd, src1, dst1)
    return _tc_final(sA, sB, degA, degB, batch.reshape(N, 1), b4, Wl, bl,
                     num_graphs)


# speedup vs baseline: 1.2923x; 1.2462x over previous
import jax
import jax.numpy as jnp
from jax import lax
from jax.experimental import pallas as pl
from jax.experimental.pallas import tpu as pltpu
from jax.experimental.pallas import tpu_sc as plsc

N = 10000
E = 320000
D = 128
NC, NS = 2, 16
NW = NC * NS
E_PER_W = E // NW
CHUNK = 80
NCH = E_PER_W // CHUNK
DEG_W = 16
R = 1000
ROWS_A = 632
ROWS_B = N - (NS - 1) * ROWS_A


def _vmesh():
    return plsc.VectorSubcoreMesh(core_axis_name="c", subcore_axis_name="s")


def _for_tile_rows(s, fn):
    @pl.when(s < NS - 1)
    def _():
        fn(pl.ds(pl.multiple_of(s * ROWS_A, 8), ROWS_A))

    @pl.when(s == NS - 1)
    def _():
        fn(pl.ds((NS - 1) * ROWS_A, ROWS_B))


def _sc_degree(dst1, init2):
    out_t = (jax.ShapeDtypeStruct((N, DEG_W), jnp.float32),) * 2

    NSLOT = 4

    @pl.kernel(out_type=out_t, mesh=_vmesh(),
               scratch_types=[pltpu.VMEM_SHARED((N, DEG_W), jnp.float32),
                              pltpu.VMEM((NSLOT, CHUNK), jnp.int32),
                              pltpu.VMEM((CHUNK, DEG_W), jnp.float32),
                              [pltpu.SemaphoreType.DMA] * NSLOT,
                              [pltpu.SemaphoreType.DMA] * NSLOT])
    def k(dst_hbm, init_hbm, oA, oB, acc, didx, onesb, dsems, ssems):
        c = lax.axis_index("c")
        s = lax.axis_index("s")
        wid = c * NS + s

        def d_async(j, slot):
            return pltpu.async_copy(
                dst_hbm.at[pl.ds(wid * E_PER_W + j * CHUNK, CHUNK)],
                didx.at[slot], dsems[slot])

        _for_tile_rows(s, lambda sl: pltpu.sync_copy(init_hbm.at[c].at[sl],
                                                     acc.at[sl]))
        pltpu.sync_copy(init_hbm.at[0].at[pl.ds(0, CHUNK)], onesb)
        plsc.subcore_barrier()

        def group(j, nslot):
            ds = [d_async(j + i, i) for i in range(nslot)]
            ss = []
            for i in range(nslot):
                ds[i].wait()
                ss.append(pltpu.async_copy(onesb, acc.at[didx.at[i]],
                                           ssems[i], add=True))
            for cp in ss:
                cp.wait()

        @pl.loop(0, NCH - 1, step=NSLOT)
        def _(j):
            group(j, NSLOT)

        group(NCH - 1, 1)
        plsc.subcore_barrier()

        @pl.when(c == 0)
        def _():
            _for_tile_rows(s, lambda sl: pltpu.sync_copy(acc.at[sl], oA.at[sl]))

        @pl.when(c == 1)
        def _():
            _for_tile_rows(s, lambda sl: pltpu.sync_copy(acc.at[sl], oB.at[sl]))

    return k(dst1, init2)


def _sc_aggregate(z, zeros_nd, src1, dst1):
    out_t = (jax.ShapeDtypeStruct((N, D), jnp.float32),) * 2
    CH = 40
    NCHU = E_PER_W // CH
    NSLOT = 5
    BODY = 25

    @pl.kernel(out_type=out_t, mesh=_vmesh(),
               scratch_types=[pltpu.VMEM_SHARED((N, D), jnp.float32),
                              pltpu.VMEM((E_PER_W,), jnp.int32),
                              pltpu.VMEM((NSLOT, CH), jnp.int32),
                              pltpu.VMEM((NSLOT, CH, D), jnp.float32),
                              [pltpu.SemaphoreType.DMA] * NSLOT,
                              [pltpu.SemaphoreType.DMA] * NSLOT,
                              [pltpu.SemaphoreType.DMA] * NSLOT])
    def k(z_hbm, zeros_hbm, src_hbm, dst_hbm, oA, oB,
          acc, sidx, didx, rows, gsems, dsems, ssems):
        c = lax.axis_index("c")
        s = lax.axis_index("s")
        wid = c * NS + s
        base = wid * E_PER_W
        i1 = pltpu.async_copy(src_hbm.at[pl.ds(base, E_PER_W)], sidx, gsems[0])

        def d_async(j, slot):
            return pltpu.async_copy(
                dst_hbm.at[pl.ds(base + j * CH, CH)], didx.at[slot],
                dsems[slot])

        @pl.when(c == 0)
        def _():
            _for_tile_rows(s, lambda sl: pltpu.sync_copy(z_hbm.at[sl],
                                                         acc.at[sl]))

        @pl.when(c == 1)
        def _():
            _for_tile_rows(s, lambda sl: pltpu.sync_copy(zeros_hbm.at[sl],
                                                         acc.at[sl]))

        i1.wait()
        plsc.subcore_barrier()

        def g_async(j, slot):
            return pltpu.async_copy(
                z_hbm.at[sidx.at[pl.ds(j * CH, CH)]], rows.at[slot],
                gsems[slot])

        def s_async(slot):
            return pltpu.async_copy(rows.at[slot], acc.at[didx.at[slot]],
                                    ssems[slot], add=True)

        @pl.loop(0, NCHU, step=BODY)
        def _(j):
            gd = {}

            def fire(k):
                slot = k % NSLOT
                gd[k] = (d_async(j + k, slot), g_async(j + k, slot))

            for k in range(NSLOT):
                fire(k)
            ss = {}
            for gi in range(BODY // NSLOT):
                b = gi * NSLOT
                for i in range(NSLOT):
                    dk, gk = gd[b + i]
                    gk.wait()
                    dk.wait()
                    ss[b + i] = s_async((b + i) % NSLOT)
                for i in range(NSLOT):
                    ss[b + i].wait()
                    if b + NSLOT + i < BODY:
                        fire(b + NSLOT + i)

        plsc.subcore_barrier()

        @pl.when(c == 0)
        def _():
            _for_tile_rows(s, lambda sl: pltpu.sync_copy(acc.at[sl], oA.at[sl]))

        @pl.when(c == 1)
        def _():
            _for_tile_rows(s, lambda sl: pltpu.sync_copy(acc.at[sl], oB.at[sl]))

    return k(z, zeros_nd, src1, dst1)


def _dinv_block(dA_ref, dB_ref):
    return lax.rsqrt(dA_ref[:, 0:1] + dB_ref[:, 0:1])


def _tc_layer1(x, W1, degA, degB):
    def body(x_ref, w_ref, dA_ref, dB_ref, o_ref):
        dinv = _dinv_block(dA_ref, dB_ref)
        u = jnp.dot(x_ref[...], w_ref[...], preferred_element_type=jnp.float32)
        o_ref[...] = dinv * u

    return pl.pallas_call(
        body,
        grid=(N // R,),
        in_specs=[pl.BlockSpec((R, D), lambda i: (i, 0)),
                  pl.BlockSpec((D, D), lambda i: (0, 0)),
                  pl.BlockSpec((R, DEG_W), lambda i: (i, 0)),
                  pl.BlockSpec((R, DEG_W), lambda i: (i, 0))],
        out_specs=pl.BlockSpec((R, D), lambda i: (i, 0)),
        out_shape=jax.ShapeDtypeStruct((N, D), jnp.float32),
    )(x, W1, degA, degB)


def _tc_layer(sA, sB, degA, degB, b_prev, W):
    def body(sA_ref, sB_ref, dA_ref, dB_ref, b_ref, w_ref, o_ref):
        dinv = _dinv_block(dA_ref, dB_ref)
        h = jnp.maximum(dinv * (sA_ref[...] + sB_ref[...]) + b_ref[...], 0.0)
        u = jnp.dot(h, w_ref[...], preferred_element_type=jnp.float32)
        o_ref[...] = dinv * u

    return pl.pallas_call(
        body,
        grid=(N // R,),
        in_specs=[pl.BlockSpec((R, D), lambda i: (i, 0)),
                  pl.BlockSpec((R, D), lambda i: (i, 0)),
                  pl.BlockSpec((R, DEG_W), lambda i: (i, 0)),
                  pl.BlockSpec((R, DEG_W), lambda i: (i, 0)),
                  pl.BlockSpec((1, D), lambda i: (0, 0)),
                  pl.BlockSpec((D, D), lambda i: (0, 0))],
        out_specs=pl.BlockSpec((R, D), lambda i: (i, 0)),
        out_shape=jax.ShapeDtypeStruct((N, D), jnp.float32),
    )(sA, sB, degA, degB, b_prev.reshape(1, D), W)


def _tc_final(sA, sB, degA, degB, batch2d, b4, Wl, bl, num_graphs):
    G = num_graphs
    n_cls = Wl.shape[1]
    steps = N // R

    def body(sA_ref, sB_ref, dA_ref, dB_ref, bt_ref, b4_ref, wl_ref, bl_ref,
             o_ref, gsum, cnt):
        i = pl.program_id(0)

        @pl.when(i == 0)
        def _():
            gsum[...] = jnp.zeros_like(gsum)
            cnt[...] = jnp.zeros_like(cnt)

        dinv = _dinv_block(dA_ref, dB_ref)
        t = dinv * (sA_ref[...] + sB_ref[...])
        gid = lax.broadcasted_iota(jnp.int32, (1, G), 1)
        oh = (bt_ref[...] == gid).astype(jnp.float32)
        gsum[...] += lax.dot_general(oh, t, (((0,), (0,)), ((), ())),
                                     preferred_element_type=jnp.float32)
        cnt[...] += lax.dot_general(oh, jnp.ones((R, 1), jnp.float32),
                                    (((0,), (0,)), ((), ())),
                                    preferred_element_type=jnp.float32)

        @pl.when(i == steps - 1)
        def _():
            gmean = gsum[...] / jnp.maximum(cnt[...], 1.0)
            o_ref[...] = jnp.dot(gmean + b4_ref[...], wl_ref[...],
                                 preferred_element_type=jnp.float32) + bl_ref[...]

    return pl.pallas_call(
        body,
        grid=(steps,),
        in_specs=[pl.BlockSpec((R, D), lambda i: (i, 0)),
                  pl.BlockSpec((R, D), lambda i: (i, 0)),
                  pl.BlockSpec((R, DEG_W), lambda i: (i, 0)),
                  pl.BlockSpec((R, DEG_W), lambda i: (i, 0)),
                  pl.BlockSpec((R, 1), lambda i: (i, 0)),
                  pl.BlockSpec((1, D), lambda i: (0, 0)),
                  pl.BlockSpec((D, n_cls), lambda i: (0, 0)),
                  pl.BlockSpec((1, n_cls), lambda i: (0, 0))],
        out_specs=pl.BlockSpec((G, n_cls), lambda i: (0, 0)),
        out_shape=jax.ShapeDtypeStruct((G, n_cls), jnp.float32),
        scratch_shapes=[pltpu.VMEM((G, D), jnp.float32),
                        pltpu.VMEM((G, 1), jnp.float32)],
    )(sA, sB, degA, degB, batch2d, b4.reshape(1, D), Wl,
      bl.reshape(1, n_cls))


def kernel(x, edge_index, batch, y, W1, b1, W2, b2, W3, b3, W4, b4, Wl, bl):
    src1 = edge_index[0]
    dst1 = edge_index[1]
    num_graphs = y.shape[0]
    init2 = jnp.concatenate([jnp.ones((1, N, DEG_W), jnp.float32),
                             jnp.zeros((1, N, DEG_W), jnp.float32)], axis=0)
    zeros_nd = jnp.zeros((N, D), jnp.float32)

    degA, degB = _sc_degree(dst1, init2)
    z = _tc_layer1(x, W1, degA, degB)
    sA, sB = _sc_aggregate(z, zeros_nd, src1, dst1)
    z = _tc_layer(sA, sB, degA, degB, b1, W2)
    sA, sB = _sc_aggregate(z, zeros_nd, src1, dst1)
    z = _tc_layer(sA, sB, degA, degB, b2, W3)
    sA, sB = _sc_aggregate(z, zeros_nd, src1, dst1)
    z = _tc_layer(sA, sB, degA, degB, b3, W4)
    sA, sB = _sc_aggregate(z, zeros_nd, src1, dst1)
    return _tc_final(sA, sB, degA, degB, batch.reshape(N, 1), b4, Wl, bl,
                     num_graphs)


# speedup vs baseline: 1.2962x; 1.0030x over previous
import jax
import jax.numpy as jnp
from jax import lax
from jax.experimental import pallas as pl
from jax.experimental.pallas import tpu as pltpu
from jax.experimental.pallas import tpu_sc as plsc

N = 10000
E = 320000
D = 128
NC, NS = 2, 16
NW = NC * NS
E_PER_W = E // NW
CHUNK = 80
NCH = E_PER_W // CHUNK
DEG_W = 16
R = 1000
ROWS_A = 632
ROWS_B = N - (NS - 1) * ROWS_A


def _vmesh():
    return plsc.VectorSubcoreMesh(core_axis_name="c", subcore_axis_name="s")


def _for_tile_rows(s, fn):
    @pl.when(s < NS - 1)
    def _():
        fn(pl.ds(pl.multiple_of(s * ROWS_A, 8), ROWS_A))

    @pl.when(s == NS - 1)
    def _():
        fn(pl.ds((NS - 1) * ROWS_A, ROWS_B))


def _sc_degree(dst1, init2):
    out_t = (jax.ShapeDtypeStruct((N, DEG_W), jnp.float32),) * 2

    NSLOT = 4

    @pl.kernel(out_type=out_t, mesh=_vmesh(),
               scratch_types=[pltpu.VMEM_SHARED((N, DEG_W), jnp.float32),
                              pltpu.VMEM((NSLOT, CHUNK), jnp.int32),
                              pltpu.VMEM((CHUNK, DEG_W), jnp.float32),
                              [pltpu.SemaphoreType.DMA] * NSLOT,
                              [pltpu.SemaphoreType.DMA] * NSLOT])
    def k(dst_hbm, init_hbm, oA, oB, acc, didx, onesb, dsems, ssems):
        c = lax.axis_index("c")
        s = lax.axis_index("s")
        wid = c * NS + s

        def d_async(j, slot):
            return pltpu.async_copy(
                dst_hbm.at[pl.ds(wid * E_PER_W + j * CHUNK, CHUNK)],
                didx.at[slot], dsems[slot])

        _for_tile_rows(s, lambda sl: pltpu.sync_copy(init_hbm.at[c].at[sl],
                                                     acc.at[sl]))
        pltpu.sync_copy(init_hbm.at[0].at[pl.ds(0, CHUNK)], onesb)
        plsc.subcore_barrier()

        def group(j, nslot):
            ds = [d_async(j + i, i) for i in range(nslot)]
            ss = []
            for i in range(nslot):
                ds[i].wait()
                ss.append(pltpu.async_copy(onesb, acc.at[didx.at[i]],
                                           ssems[i], add=True))
            for cp in ss:
                cp.wait()

        @pl.loop(0, NCH - 1, step=NSLOT)
        def _(j):
            group(j, NSLOT)

        group(NCH - 1, 1)
        plsc.subcore_barrier()

        @pl.when(c == 0)
        def _():
            _for_tile_rows(s, lambda sl: pltpu.sync_copy(acc.at[sl], oA.at[sl]))

        @pl.when(c == 1)
        def _():
            _for_tile_rows(s, lambda sl: pltpu.sync_copy(acc.at[sl], oB.at[sl]))

    return k(dst1, init2)


def _sc_aggregate(z, zeros_nd, src1, dst1):
    out_t = (jax.ShapeDtypeStruct((N, D), jnp.float32),) * 2
    CH = 40
    NCHU = E_PER_W // CH
    NSLOT = 5
    BODY = 50

    @pl.kernel(out_type=out_t, mesh=_vmesh(),
               scratch_types=[pltpu.VMEM_SHARED((N, D), jnp.float32),
                              pltpu.VMEM((E_PER_W,), jnp.int32),
                              pltpu.VMEM((NSLOT, CH), jnp.int32),
                              pltpu.VMEM((NSLOT, CH, D), jnp.float32),
                              [pltpu.SemaphoreType.DMA] * NSLOT,
                              [pltpu.SemaphoreType.DMA] * NSLOT,
                              [pltpu.SemaphoreType.DMA] * NSLOT])
    def k(z_hbm, zeros_hbm, src_hbm, dst_hbm, oA, oB,
          acc, sidx, didx, rows, gsems, dsems, ssems):
        c = lax.axis_index("c")
        s = lax.axis_index("s")
        wid = c * NS + s
        base = wid * E_PER_W
        i1 = pltpu.async_copy(src_hbm.at[pl.ds(base, E_PER_W)], sidx, gsems[0])

        def d_async(j, slot):
            return pltpu.async_copy(
                dst_hbm.at[pl.ds(base + j * CH, CH)], didx.at[slot],
                dsems[slot])

        @pl.when(c == 0)
        def _():
            _for_tile_rows(s, lambda sl: pltpu.sync_copy(z_hbm.at[sl],
                                                         acc.at[sl]))

        @pl.when(c == 1)
        def _():
            _for_tile_rows(s, lambda sl: pltpu.sync_copy(zeros_hbm.at[sl],
                                                         acc.at[sl]))

        i1.wait()
        plsc.subcore_barrier()

        def g_async(j, slot):
            return pltpu.async_copy(
                z_hbm.at[sidx.at[pl.ds(j * CH, CH)]], rows.at[slot],
                gsems[slot])

        def s_async(slot):
            return pltpu.async_copy(rows.at[slot], acc.at[didx.at[slot]],
                                    ssems[slot], add=True)

        @pl.loop(0, NCHU, step=BODY)
        def _(j):
            gd = {}

            def fire(k):
                slot = k % NSLOT
                gd[k] = (d_async(j + k, slot), g_async(j + k, slot))

            for k in range(NSLOT):
                fire(k)
            ss = {}
            for gi in range(BODY // NSLOT):
                b = gi * NSLOT
                for i in range(NSLOT):
                    dk, gk = gd[b + i]
                    gk.wait()
                    dk.wait()
                    ss[b + i] = s_async((b + i) % NSLOT)
                for i in range(NSLOT):
                    ss[b + i].wait()
                    if b + NSLOT + i < BODY:
                        fire(b + NSLOT + i)

        plsc.subcore_barrier()

        @pl.when(c == 0)
        def _():
            _for_tile_rows(s, lambda sl: pltpu.sync_copy(acc.at[sl], oA.at[sl]))

        @pl.when(c == 1)
        def _():
            _for_tile_rows(s, lambda sl: pltpu.sync_copy(acc.at[sl], oB.at[sl]))

    return k(z, zeros_nd, src1, dst1)


def _dinv_block(dA_ref, dB_ref):
    return lax.rsqrt(dA_ref[:, 0:1] + dB_ref[:, 0:1])


def _tc_layer1(x, W1, degA, degB):
    def body(x_ref, w_ref, dA_ref, dB_ref, o_ref):
        dinv = _dinv_block(dA_ref, dB_ref)
        u = jnp.dot(x_ref[...], w_ref[...], preferred_element_type=jnp.float32)
        o_ref[...] = dinv * u

    return pl.pallas_call(
        body,
        grid=(N // R,),
        in_specs=[pl.BlockSpec((R, D), lambda i: (i, 0)),
                  pl.BlockSpec((D, D), lambda i: (0, 0)),
                  pl.BlockSpec((R, DEG_W), lambda i: (i, 0)),
                  pl.BlockSpec((R, DEG_W), lambda i: (i, 0))],
        out_specs=pl.BlockSpec((R, D), lambda i: (i, 0)),
        out_shape=jax.ShapeDtypeStruct((N, D), jnp.float32),
    )(x, W1, degA, degB)


def _tc_layer(sA, sB, degA, degB, b_prev, W):
    def body(sA_ref, sB_ref, dA_ref, dB_ref, b_ref, w_ref, o_ref):
        dinv = _dinv_block(dA_ref, dB_ref)
        h = jnp.maximum(dinv * (sA_ref[...] + sB_ref[...]) + b_ref[...], 0.0)
        u = jnp.dot(h, w_ref[...], preferred_element_type=jnp.float32)
        o_ref[...] = dinv * u

    return pl.pallas_call(
        body,
        grid=(N // R,),
        in_specs=[pl.BlockSpec((R, D), lambda i: (i, 0)),
                  pl.BlockSpec((R, D), lambda i: (i, 0)),
                  pl.BlockSpec((R, DEG_W), lambda i: (i, 0)),
                  pl.BlockSpec((R, DEG_W), lambda i: (i, 0)),
                  pl.BlockSpec((1, D), lambda i: (0, 0)),
                  pl.BlockSpec((D, D), lambda i: (0, 0))],
        out_specs=pl.BlockSpec((R, D), lambda i: (i, 0)),
        out_shape=jax.ShapeDtypeStruct((N, D), jnp.float32),
    )(sA, sB, degA, degB, b_prev.reshape(1, D), W)


def _tc_final(sA, sB, degA, degB, batch2d, b4, Wl, bl, num_graphs):
    G = num_graphs
    n_cls = Wl.shape[1]
    steps = N // R

    def body(sA_ref, sB_ref, dA_ref, dB_ref, bt_ref, b4_ref, wl_ref, bl_ref,
             o_ref, gsum, cnt):
        i = pl.program_id(0)

        @pl.when(i == 0)
        def _():
            gsum[...] = jnp.zeros_like(gsum)
            cnt[...] = jnp.zeros_like(cnt)

        dinv = _dinv_block(dA_ref, dB_ref)
        t = dinv * (sA_ref[...] + sB_ref[...])
        gid = lax.broadcasted_iota(jnp.int32, (1, G), 1)
        oh = (bt_ref[...] == gid).astype(jnp.float32)
        gsum[...] += lax.dot_general(oh, t, (((0,), (0,)), ((), ())),
                                     preferred_element_type=jnp.float32)
        cnt[...] += lax.dot_general(oh, jnp.ones((R, 1), jnp.float32),
                                    (((0,), (0,)), ((), ())),
                                    preferred_element_type=jnp.float32)

        @pl.when(i == steps - 1)
        def _():
            gmean = gsum[...] / jnp.maximum(cnt[...], 1.0)
            o_ref[...] = jnp.dot(gmean + b4_ref[...], wl_ref[...],
                                 preferred_element_type=jnp.float32) + bl_ref[...]

    return pl.pallas_call(
        body,
        grid=(steps,),
        in_specs=[pl.BlockSpec((R, D), lambda i: (i, 0)),
                  pl.BlockSpec((R, D), lambda i: (i, 0)),
                  pl.BlockSpec((R, DEG_W), lambda i: (i, 0)),
                  pl.BlockSpec((R, DEG_W), lambda i: (i, 0)),
                  pl.BlockSpec((R, 1), lambda i: (i, 0)),
                  pl.BlockSpec((1, D), lambda i: (0, 0)),
                  pl.BlockSpec((D, n_cls), lambda i: (0, 0)),
                  pl.BlockSpec((1, n_cls), lambda i: (0, 0))],
        out_specs=pl.BlockSpec((G, n_cls), lambda i: (0, 0)),
        out_shape=jax.ShapeDtypeStruct((G, n_cls), jnp.float32),
        scratch_shapes=[pltpu.VMEM((G, D), jnp.float32),
                        pltpu.VMEM((G, 1), jnp.float32)],
    )(sA, sB, degA, degB, batch2d, b4.reshape(1, D), Wl,
      bl.reshape(1, n_cls))


def kernel(x, edge_index, batch, y, W1, b1, W2, b2, W3, b3, W4, b4, Wl, bl):
    src1 = edge_index[0]
    dst1 = edge_index[1]
    num_graphs = y.shape[0]
    init2 = jnp.concatenate([jnp.ones((1, N, DEG_W), jnp.float32),
                             jnp.zeros((1, N, DEG_W), jnp.float32)], axis=0)
    zeros_nd = jnp.zeros((N, D), jnp.float32)

    degA, degB = _sc_degree(dst1, init2)
    z = _tc_layer1(x, W1, degA, degB)
    sA, sB = _sc_aggregate(z, zeros_nd, src1, dst1)
    z = _tc_layer(sA, sB, degA, degB, b1, W2)
    sA, sB = _sc_aggregate(z, zeros_nd, src1, dst1)
    z = _tc_layer(sA, sB, degA, degB, b2, W3)
    sA, sB = _sc_aggregate(z, zeros_nd, src1, dst1)
    z = _tc_layer(sA, sB, degA, degB, b3, W4)
    sA, sB = _sc_aggregate(z, zeros_nd, src1, dst1)
    return _tc_final(sA, sB, degA, degB, batch.reshape(N, 1), b4, Wl, bl,
                     num_graphs)


# speedup vs baseline: 1.3112x; 1.0116x over previous
import jax
import jax.numpy as jnp
from jax import lax
from jax.experimental import pallas as pl
from jax.experimental.pallas import tpu as pltpu
from jax.experimental.pallas import tpu_sc as plsc

N = 10000
E = 320000
D = 128
NC, NS = 2, 16
NW = NC * NS
E_PER_W = E // NW
CHUNK = 80
NCH = E_PER_W // CHUNK
DEG_W = 16
R = 1000
ROWS_A = 632
ROWS_B = N - (NS - 1) * ROWS_A


def _vmesh():
    return plsc.VectorSubcoreMesh(core_axis_name="c", subcore_axis_name="s")


def _for_tile_rows(s, fn):
    @pl.when(s < NS - 1)
    def _():
        fn(pl.ds(pl.multiple_of(s * ROWS_A, 8), ROWS_A))

    @pl.when(s == NS - 1)
    def _():
        fn(pl.ds((NS - 1) * ROWS_A, ROWS_B))


def _sc_degree(dst1, init2):
    out_t = (jax.ShapeDtypeStruct((N, DEG_W), jnp.float32),) * 2

    NSLOT = 5
    BODY = 25

    @pl.kernel(out_type=out_t, mesh=_vmesh(),
               scratch_types=[pltpu.VMEM_SHARED((N, DEG_W), jnp.float32),
                              pltpu.VMEM((NSLOT, CHUNK), jnp.int32),
                              pltpu.VMEM((CHUNK, DEG_W), jnp.float32),
                              [pltpu.SemaphoreType.DMA] * NSLOT,
                              [pltpu.SemaphoreType.DMA] * NSLOT])
    def k(dst_hbm, init_hbm, oA, oB, acc, didx, onesb, dsems, ssems):
        c = lax.axis_index("c")
        s = lax.axis_index("s")
        wid = c * NS + s

        def d_async(j, slot):
            return pltpu.async_copy(
                dst_hbm.at[pl.ds(wid * E_PER_W + j * CHUNK, CHUNK)],
                didx.at[slot], dsems[slot])

        _for_tile_rows(s, lambda sl: pltpu.sync_copy(init_hbm.at[c].at[sl],
                                                     acc.at[sl]))
        pltpu.sync_copy(init_hbm.at[0].at[pl.ds(0, CHUNK)], onesb)
        plsc.subcore_barrier()

        @pl.loop(0, NCH - NCH % BODY, step=BODY)
        def _(j):
            gd = {}

            def fire(k):
                gd[k] = d_async(j + k, k % NSLOT)

            for k in range(NSLOT):
                fire(k)
            ss = {}
            for gi in range(BODY // NSLOT):
                b = gi * NSLOT
                for i in range(NSLOT):
                    gd[b + i].wait()
                    ss[b + i] = pltpu.async_copy(
                        onesb, acc.at[didx.at[(b + i) % NSLOT]],
                        ssems[(b + i) % NSLOT], add=True)
                for i in range(NSLOT):
                    ss[b + i].wait()
                    if b + NSLOT + i < BODY:
                        fire(b + NSLOT + i)

        plsc.subcore_barrier()

        @pl.when(c == 0)
        def _():
            _for_tile_rows(s, lambda sl: pltpu.sync_copy(acc.at[sl], oA.at[sl]))

        @pl.when(c == 1)
        def _():
            _for_tile_rows(s, lambda sl: pltpu.sync_copy(acc.at[sl], oB.at[sl]))

    return k(dst1, init2)


def _sc_aggregate(z, zeros_nd, src1, dst1):
    out_t = (jax.ShapeDtypeStruct((N, D), jnp.float32),) * 2
    CH = 40
    NCHU = E_PER_W // CH
    NSLOT = 5
    BODY = 50

    @pl.kernel(out_type=out_t, mesh=_vmesh(),
               scratch_types=[pltpu.VMEM_SHARED((N, D), jnp.float32),
                              pltpu.VMEM((E_PER_W,), jnp.int32),
                              pltpu.VMEM((NSLOT, CH), jnp.int32),
                              pltpu.VMEM((NSLOT, CH, D), jnp.float32),
                              [pltpu.SemaphoreType.DMA] * NSLOT,
                              [pltpu.SemaphoreType.DMA] * NSLOT,
                              [pltpu.SemaphoreType.DMA] * NSLOT])
    def k(z_hbm, zeros_hbm, src_hbm, dst_hbm, oA, oB,
          acc, sidx, didx, rows, gsems, dsems, ssems):
        c = lax.axis_index("c")
        s = lax.axis_index("s")
        wid = c * NS + s
        base = wid * E_PER_W
        i1 = pltpu.async_copy(src_hbm.at[pl.ds(base, E_PER_W)], sidx, gsems[0])

        def d_async(j, slot):
            return pltpu.async_copy(
                dst_hbm.at[pl.ds(base + j * CH, CH)], didx.at[slot],
                dsems[slot])

        @pl.when(c == 0)
        def _():
            _for_tile_rows(s, lambda sl: pltpu.sync_copy(z_hbm.at[sl],
                                                         acc.at[sl]))

        @pl.when(c == 1)
        def _():
            _for_tile_rows(s, lambda sl: pltpu.sync_copy(zeros_hbm.at[sl],
                                                         acc.at[sl]))

        i1.wait()
        plsc.subcore_barrier()

        def g_async(j, slot):
            return pltpu.async_copy(
                z_hbm.at[sidx.at[pl.ds(j * CH, CH)]], rows.at[slot],
                gsems[slot])

        def s_async(slot):
            return pltpu.async_copy(rows.at[slot], acc.at[didx.at[slot]],
                                    ssems[slot], add=True)

        @pl.loop(0, NCHU, step=BODY)
        def _(j):
            gd = {}

            def fire(k):
                slot = k % NSLOT
                gd[k] = (d_async(j + k, slot), g_async(j + k, slot))

            for k in range(NSLOT):
                fire(k)
            ss = {}
            for gi in range(BODY // NSLOT):
                b = gi * NSLOT
                for i in range(NSLOT):
                    dk, gk = gd[b + i]
                    gk.wait()
                    dk.wait()
                    ss[b + i] = s_async((b + i) % NSLOT)
                for i in range(NSLOT):
                    ss[b + i].wait()
                    if b + NSLOT + i < BODY:
                        fire(b + NSLOT + i)

        plsc.subcore_barrier()

        @pl.when(c == 0)
        def _():
            _for_tile_rows(s, lambda sl: pltpu.sync_copy(acc.at[sl], oA.at[sl]))

        @pl.when(c == 1)
        def _():
            _for_tile_rows(s, lambda sl: pltpu.sync_copy(acc.at[sl], oB.at[sl]))

    return k(z, zeros_nd, src1, dst1)


def _dinv_block(dA_ref, dB_ref):
    return lax.rsqrt(dA_ref[:, 0:1] + dB_ref[:, 0:1])


def _tc_layer1(x, W1, degA, degB):
    def body(x_ref, w_ref, dA_ref, dB_ref, o_ref):
        dinv = _dinv_block(dA_ref, dB_ref)
        u = jnp.dot(x_ref[...], w_ref[...], preferred_element_type=jnp.float32)
        o_ref[...] = dinv * u

    return pl.pallas_call(
        body,
        grid=(N // R,),
        in_specs=[pl.BlockSpec((R, D), lambda i: (i, 0)),
                  pl.BlockSpec((D, D), lambda i: (0, 0)),
                  pl.BlockSpec((R, DEG_W), lambda i: (i, 0)),
                  pl.BlockSpec((R, DEG_W), lambda i: (i, 0))],
        out_specs=pl.BlockSpec((R, D), lambda i: (i, 0)),
        out_shape=jax.ShapeDtypeStruct((N, D), jnp.float32),
    )(x, W1, degA, degB)


def _tc_layer(sA, sB, degA, degB, b_prev, W):
    def body(sA_ref, sB_ref, dA_ref, dB_ref, b_ref, w_ref, o_ref):
        dinv = _dinv_block(dA_ref, dB_ref)
        h = jnp.maximum(dinv * (sA_ref[...] + sB_ref[...]) + b_ref[...], 0.0)
        u = jnp.dot(h, w_ref[...], preferred_element_type=jnp.float32)
        o_ref[...] = dinv * u

    return pl.pallas_call(
        body,
        grid=(N // R,),
        in_specs=[pl.BlockSpec((R, D), lambda i: (i, 0)),
                  pl.BlockSpec((R, D), lambda i: (i, 0)),
                  pl.BlockSpec((R, DEG_W), lambda i: (i, 0)),
                  pl.BlockSpec((R, DEG_W), lambda i: (i, 0)),
                  pl.BlockSpec((1, D), lambda i: (0, 0)),
                  pl.BlockSpec((D, D), lambda i: (0, 0))],
        out_specs=pl.BlockSpec((R, D), lambda i: (i, 0)),
        out_shape=jax.ShapeDtypeStruct((N, D), jnp.float32),
    )(sA, sB, degA, degB, b_prev.reshape(1, D), W)


def _tc_final(sA, sB, degA, degB, batch2d, b4, Wl, bl, num_graphs):
    G = num_graphs
    n_cls = Wl.shape[1]
    steps = N // R

    def body(sA_ref, sB_ref, dA_ref, dB_ref, bt_ref, b4_ref, wl_ref, bl_ref,
             o_ref, gsum, cnt):
        i = pl.program_id(0)

        @pl.when(i == 0)
        def _():
            gsum[...] = jnp.zeros_like(gsum)
            cnt[...] = jnp.zeros_like(cnt)

        dinv = _dinv_block(dA_ref, dB_ref)
        t = dinv * (sA_ref[...] + sB_ref[...])
        gid = lax.broadcasted_iota(jnp.int32, (1, G), 1)
        oh = (bt_ref[...] == gid).astype(jnp.float32)
        gsum[...] += lax.dot_general(oh, t, (((0,), (0,)), ((), ())),
                                     preferred_element_type=jnp.float32)
        cnt[...] += lax.dot_general(oh, jnp.ones((R, 1), jnp.float32),
                                    (((0,), (0,)), ((), ())),
                                    preferred_element_type=jnp.float32)

        @pl.when(i == steps - 1)
        def _():
            gmean = gsum[...] / jnp.maximum(cnt[...], 1.0)
            o_ref[...] = jnp.dot(gmean + b4_ref[...], wl_ref[...],
                                 preferred_element_type=jnp.float32) + bl_ref[...]

    return pl.pallas_call(
        body,
        grid=(steps,),
        in_specs=[pl.BlockSpec((R, D), lambda i: (i, 0)),
                  pl.BlockSpec((R, D), lambda i: (i, 0)),
                  pl.BlockSpec((R, DEG_W), lambda i: (i, 0)),
                  pl.BlockSpec((R, DEG_W), lambda i: (i, 0)),
                  pl.BlockSpec((R, 1), lambda i: (i, 0)),
                  pl.BlockSpec((1, D), lambda i: (0, 0)),
                  pl.BlockSpec((D, n_cls), lambda i: (0, 0)),
                  pl.BlockSpec((1, n_cls), lambda i: (0, 0))],
        out_specs=pl.BlockSpec((G, n_cls), lambda i: (0, 0)),
        out_shape=jax.ShapeDtypeStruct((G, n_cls), jnp.float32),
        scratch_shapes=[pltpu.VMEM((G, D), jnp.float32),
                        pltpu.VMEM((G, 1), jnp.float32)],
    )(sA, sB, degA, degB, batch2d, b4.reshape(1, D), Wl,
      bl.reshape(1, n_cls))


def kernel(x, edge_index, batch, y, W1, b1, W2, b2, W3, b3, W4, b4, Wl, bl):
    src1 = edge_index[0]
    dst1 = edge_index[1]
    num_graphs = y.shape[0]
    init2 = jnp.concatenate([jnp.ones((1, N, DEG_W), jnp.float32),
                             jnp.zeros((1, N, DEG_W), jnp.float32)], axis=0)
    zeros_nd = jnp.zeros((N, D), jnp.float32)

    degA, degB = _sc_degree(dst1, init2)
    z = _tc_layer1(x, W1, degA, degB)
    sA, sB = _sc_aggregate(z, zeros_nd, src1, dst1)
    z = _tc_layer(sA, sB, degA, degB, b1, W2)
    sA, sB = _sc_aggregate(z, zeros_nd, src1, dst1)
    z = _tc_layer(sA, sB, degA, degB, b2, W3)
    sA, sB = _sc_aggregate(z, zeros_nd, src1, dst1)
    z = _tc_layer(sA, sB, degA, degB, b3, W4)
    sA, sB = _sc_aggregate(z, zeros_nd, src1, dst1)
    return _tc_final(sA, sB, degA, degB, batch.reshape(N, 1), b4, Wl, bl,
                     num_graphs)
